# Initial kernel scaffold; baseline (speedup 1.0000x reference)
import numpy as np
import jax
import jax.numpy as jnp
from jax import lax
from jax.experimental import pallas as pl
from jax.experimental.pallas import tpu as pltpu

_N = 20000
_ROWS = 160
_NP = _ROWS * 128
_K = 5000
_TOPK = 750
_THR = 0.3
_CONF = 0.05
_NEG = jnp.float32(-jnp.inf)
_LO0 = int(np.float32(_CONF).view(np.int32))
_HI0 = int(np.float32(1.0).view(np.int32))


def _nms_body(loc_ref, pri_ref, sc_ref, out_ref,
              x1_ref, y1_ref, x2_ref, y2_ref, ar_ref, s_ref, idx_ref):
    lx = loc_ref[0]; ly = loc_ref[1]; lw = loc_ref[2]; lh = loc_ref[3]
    pcx = pri_ref[0]; pcy = pri_ref[1]; pw = pri_ref[2]; ph = pri_ref[3]
    cx = pcx + (lx * 0.1) * pw
    cy = pcy + (ly * 0.1) * ph
    w = pw * jnp.exp(lw * 0.2)
    h = ph * jnp.exp(lh * 0.2)
    x1 = cx - w / 2.0
    y1 = cy - h / 2.0
    x2 = w + x1
    y2 = h + y1
    area = (x2 - x1) * (y2 - y1)

    r = lax.broadcasted_iota(jnp.int32, (_ROWS, 128), 0)
    c = lax.broadcasted_iota(jnp.int32, (_ROWS, 128), 1)
    idx = r * 128 + c

    scores = sc_ref[...]
    valid = scores > _CONF
    key = jnp.where(valid, scores, jnp.float32(-1.0))
    key_i = lax.bitcast_convert_type(key, jnp.int32)
    nvalid = jnp.sum(valid.astype(jnp.int32))

    def bs_val(_, lohi):
        lo, hi = lohi
        mid = (lo + hi) // 2
        feas = jnp.sum((key_i >= mid).astype(jnp.int32)) >= _K
        return (jnp.where(feas, mid, lo), jnp.where(feas, hi, mid))

    vstar, _ = lax.fori_loop(0, 28, bs_val, (jnp.int32(_LO0), jnp.int32(_HI0)))
    n_gt = jnp.sum((key_i > vstar).astype(jnp.int32))
    m_need = _K - n_gt
    emask = key_i == vstar

    def bs_idx(_, lohi):
        lo, hi = lohi
        mid = (lo + hi) // 2
        feas = jnp.sum((emask & (idx >= mid)).astype(jnp.int32)) >= m_need
        return (jnp.where(feas, mid, lo), jnp.where(feas, hi, mid))

    tstar, _ = lax.fori_loop(0, 16, bs_idx, (jnp.int32(0), jnp.int32(_NP)))
    elig = jnp.where(nvalid > _K,
                     (key_i > vstar) | (emask & (idx >= tstar)),
                     valid)

    x1_ref[...] = x1; y1_ref[...] = y1
    x2_ref[...] = x2; y2_ref[...] = y2
    ar_ref[...] = area
    idx_ref[...] = idx
    s0 = jnp.where(elig, scores, _NEG)
    s_ref[...] = s0
    out_ref[...] = jnp.zeros((_TOPK, 1, 128), jnp.float32)

    def cond(carry):
        j, m = carry
        return (j < _TOPK) & (m > _NEG)

    def body(carry):
        j, m = carry
        s = s_ref[...]
        idxv = idx_ref[...]
        pick = jnp.max(jnp.where(s == m, idxv, -1))
        sel = idxv == pick
        bx1 = x1_ref[...]; by1 = y1_ref[...]
        bx2 = x2_ref[...]; by2 = y2_ref[...]
        ar = ar_ref[...]
        px1 = jnp.max(jnp.where(sel, bx1, _NEG))
        py1 = jnp.max(jnp.where(sel, by1, _NEG))
        px2 = jnp.max(jnp.where(sel, bx2, _NEG))
        py2 = jnp.max(jnp.where(sel, by2, _NEG))
        par = jnp.max(jnp.where(sel, ar, _NEG))
        xx1 = jnp.maximum(bx1, px1); yy1 = jnp.maximum(by1, py1)
        xx2 = jnp.minimum(bx2, px2); yy2 = jnp.minimum(by2, py2)
        ww = jnp.clip(xx2 - xx1, 0.0, None)
        hh = jnp.clip(yy2 - yy1, 0.0, None)
        inter = ww * hh
        union = ar - inter + par
        iou = inter / jnp.maximum(union, 1e-12)
        s_new = jnp.where((iou <= _THR) & (idxv != pick), s, _NEG)
        s_ref[...] = s_new
        c128 = lax.broadcasted_iota(jnp.int32, (1, 128), 1)
        row = jnp.where(c128 == 0, m, jnp.float32(0.0))
        row = jnp.where(c128 == 1, px1, row)
        row = jnp.where(c128 == 2, py1, row)
        row = jnp.where(c128 == 3, px2, row)
        row = jnp.where(c128 == 4, py2, row)
        out_ref[j] = row
        return (j + 1, jnp.max(s_new))

    m0 = jnp.max(s0)
    lax.while_loop(cond, body, (jnp.int32(0), m0))


_SCRATCHES = [pltpu.VMEM((_ROWS, 128), jnp.float32)] * 6 + \
             [pltpu.VMEM((_ROWS, 128), jnp.int32)]


def _run_nms(loc, pri, sc, interpret=False):
    return pl.pallas_call(
        _nms_body,
        out_shape=jax.ShapeDtypeStruct((_TOPK, 1, 128), jnp.float32),
        scratch_shapes=_SCRATCHES,
        interpret=interpret,
    )(loc, pri, sc)


@jax.jit
def kernel(loc_data, conf_data, prior_data):
    loc = loc_data.reshape(_N, 4).T
    pri = prior_data.T
    loc = jnp.pad(loc, ((0, 0), (0, _NP - _N))).reshape(4, _ROWS, 128)
    pri = jnp.pad(pri, ((0, 0), (0, _NP - _N))).reshape(4, _ROWS, 128)
    sc = jnp.pad(conf_data[:, 1], (0, _NP - _N)).reshape(_ROWS, 128)
    out = _run_nms(loc, pri, sc)
    cls1 = out[:, 0, :5].reshape(1, 1, _TOPK, 5)
    zero = jnp.zeros((1, 1, _TOPK, 5), jnp.float32)
    return jnp.concatenate([zero, cls1], axis=1)


# speedup vs baseline: 19.0133x; 19.0133x over previous
import numpy as np
import jax
import jax.numpy as jnp
from jax import lax
from jax.experimental import pallas as pl
from jax.experimental.pallas import tpu as pltpu

_N = 20000
_ROWS = 160
_NP = _ROWS * 128
_K = 5000
_TOPK = 750
_THR = 0.3
_CONF = 0.05
_NEG = float("-inf")
_LO0 = int(np.float32(_CONF).view(np.int32))
_HI0 = int(np.float32(1.0).view(np.int32))


def _nms_body(loc_ref, pri_ref, sc_ref, out_ref,
              x1_ref, y1_ref, x2_ref, y2_ref, ar_ref, s_ref, idx_ref):
    lx = loc_ref[0]; ly = loc_ref[1]; lw = loc_ref[2]; lh = loc_ref[3]
    pcx = pri_ref[0]; pcy = pri_ref[1]; pw = pri_ref[2]; ph = pri_ref[3]
    cx = pcx + (lx * 0.1) * pw
    cy = pcy + (ly * 0.1) * ph
    w = pw * jnp.exp(lw * 0.2)
    h = ph * jnp.exp(lh * 0.2)
    x1 = cx - w / 2.0
    y1 = cy - h / 2.0
    x2 = w + x1
    y2 = h + y1
    area = (x2 - x1) * (y2 - y1)

    r = lax.broadcasted_iota(jnp.int32, (_ROWS, 128), 0)
    c = lax.broadcasted_iota(jnp.int32, (_ROWS, 128), 1)
    idx = r * 128 + c

    scores = sc_ref[...]
    valid = scores > _CONF
    key = jnp.where(valid, scores, jnp.float32(-1.0))
    key_i = lax.bitcast_convert_type(key, jnp.int32)
    nvalid = jnp.sum(valid.astype(jnp.int32))

    def bs_val(_, lohi):
        lo, hi = lohi
        mid = (lo + hi) // 2
        feas = jnp.sum((key_i >= mid).astype(jnp.int32)) >= _K
        return (jnp.where(feas, mid, lo), jnp.where(feas, hi, mid))

    vstar, _ = lax.fori_loop(0, 28, bs_val, (jnp.int32(_LO0), jnp.int32(_HI0)))
    n_gt = jnp.sum((key_i > vstar).astype(jnp.int32))
    m_need = _K - n_gt
    emask = key_i == vstar

    def bs_idx(_, lohi):
        lo, hi = lohi
        mid = (lo + hi) // 2
        feas = jnp.sum((emask & (idx >= mid)).astype(jnp.int32)) >= m_need
        return (jnp.where(feas, mid, lo), jnp.where(feas, hi, mid))

    tstar, _ = lax.fori_loop(0, 16, bs_idx, (jnp.int32(0), jnp.int32(_NP)))
    big = nvalid > _K
    elig_top = (key_i > vstar) | (emask & (idx >= tstar))
    elig = (big & elig_top) | (jnp.logical_not(big) & valid)

    x1_ref[...] = x1; y1_ref[...] = y1
    x2_ref[...] = x2; y2_ref[...] = y2
    ar_ref[...] = area
    idx_ref[...] = idx
    s0 = jnp.where(elig, scores, _NEG)
    s_ref[...] = s0
    out_ref[...] = jnp.zeros((_TOPK, 1, 128), jnp.float32)

    def cond(carry):
        j, m = carry
        return (j < _TOPK) & (m > _NEG)

    def body(carry):
        j, m = carry
        s = s_ref[...]
        idxv = idx_ref[...]
        pick = jnp.max(jnp.where(s == m, idxv, -1))
        sel = idxv == pick
        bx1 = x1_ref[...]; by1 = y1_ref[...]
        bx2 = x2_ref[...]; by2 = y2_ref[...]
        ar = ar_ref[...]
        px1 = jnp.max(jnp.where(sel, bx1, _NEG))
        py1 = jnp.max(jnp.where(sel, by1, _NEG))
        px2 = jnp.max(jnp.where(sel, bx2, _NEG))
        py2 = jnp.max(jnp.where(sel, by2, _NEG))
        par = jnp.max(jnp.where(sel, ar, _NEG))
        xx1 = jnp.maximum(bx1, px1); yy1 = jnp.maximum(by1, py1)
        xx2 = jnp.minimum(bx2, px2); yy2 = jnp.minimum(by2, py2)
        ww = jnp.clip(xx2 - xx1, 0.0, None)
        hh = jnp.clip(yy2 - yy1, 0.0, None)
        inter = ww * hh
        union = ar - inter + par
        iou = inter / jnp.maximum(union, 1e-12)
        s_new = jnp.where((iou <= _THR) & (idxv != pick), s, _NEG)
        s_ref[...] = s_new
        c128 = lax.broadcasted_iota(jnp.int32, (1, 128), 1)
        row = jnp.where(c128 == 0, m, jnp.float32(0.0))
        row = jnp.where(c128 == 1, px1, row)
        row = jnp.where(c128 == 2, py1, row)
        row = jnp.where(c128 == 3, px2, row)
        row = jnp.where(c128 == 4, py2, row)
        out_ref[j] = row
        return (j + 1, jnp.max(s_new))

    m0 = jnp.max(s0)
    lax.while_loop(cond, body, (jnp.int32(0), m0))


_SCRATCHES = [pltpu.VMEM((_ROWS, 128), jnp.float32)] * 6 + \
             [pltpu.VMEM((_ROWS, 128), jnp.int32)]


def _run_nms(loc, pri, sc, interpret=False):
    return pl.pallas_call(
        _nms_body,
        out_shape=jax.ShapeDtypeStruct((_TOPK, 1, 128), jnp.float32),
        scratch_shapes=_SCRATCHES,
        interpret=interpret,
    )(loc, pri, sc)


@jax.jit
def kernel(loc_data, conf_data, prior_data):
    loc = loc_data.reshape(_N, 4).T
    pri = prior_data.T
    loc = jnp.pad(loc, ((0, 0), (0, _NP - _N))).reshape(4, _ROWS, 128)
    pri = jnp.pad(pri, ((0, 0), (0, _NP - _N))).reshape(4, _ROWS, 128)
    sc = jnp.pad(conf_data[:, 1], (0, _NP - _N)).reshape(_ROWS, 128)
    out = _run_nms(loc, pri, sc)
    cls1 = out[:, 0, :5].reshape(1, 1, _TOPK, 5)
    zero = jnp.zeros((1, 1, _TOPK, 5), jnp.float32)
    return jnp.concatenate([zero, cls1], axis=1)


# speedup vs baseline: 22.1274x; 1.1638x over previous
import numpy as np
import jax
import jax.numpy as jnp
from jax import lax
from jax.experimental import pallas as pl
from jax.experimental.pallas import tpu as pltpu

_N = 20000
_ROWS = 160
_NP = _ROWS * 128
_DR = 48
_K = 5000
_TOPK = 750
_THR = 0.3
_CONF = 0.05
_NEG = float("-inf")
_LO0 = int(np.float32(_CONF).view(np.int32))
_HI0 = int(np.float32(1.0).view(np.int32))
_DN = (((1,), (0,)), ((), ()))


def _lane_prefix(x):
    lane = lax.broadcasted_iota(jnp.int32, x.shape, 1)
    y = x
    for k in (1, 2, 4, 8, 16, 32, 64):
        y = y + jnp.where(lane >= k, pltpu.roll(y, k, axis=1), 0.0)
    return y


def _nms_body(lx_ref, ly_ref, lw_ref, lh_ref,
              pcx_ref, pcy_ref, pw_ref, ph_ref, sc_ref, out_ref,
              ef_ref, rk_ref,
              s_ref, x1_ref, y1_ref, x2_ref, y2_ref, idx_ref, fl_ref, ar_ref):
    scores = sc_ref[...]
    valid = scores > _CONF
    key = jnp.where(valid, scores, jnp.float32(-1.0))
    key_i = lax.bitcast_convert_type(key, jnp.int32)
    nvalid = jnp.sum(valid.astype(jnp.int32))
    r2 = lax.broadcasted_iota(jnp.int32, (_ROWS, 128), 0)
    c2 = lax.broadcasted_iota(jnp.int32, (_ROWS, 128), 1)
    idx2 = r2 * 128 + c2

    def bs_val(_, lohi):
        lo, hi = lohi
        mid = (lo + hi) // 2
        feas = jnp.sum((key_i >= mid).astype(jnp.int32)) >= _K
        return (jnp.where(feas, mid, lo), jnp.where(feas, hi, mid))

    vstar, _ = lax.fori_loop(0, 28, bs_val, (jnp.int32(_LO0), jnp.int32(_HI0)))
    n_gt = jnp.sum((key_i > vstar).astype(jnp.int32))
    m_need = _K - n_gt
    emask = key_i == vstar

    def bs_idx(_, lohi):
        lo, hi = lohi
        mid = (lo + hi) // 2
        feas = jnp.sum((emask & (idx2 >= mid)).astype(jnp.int32)) >= m_need
        return (jnp.where(feas, mid, lo), jnp.where(feas, hi, mid))

    tstar, _ = lax.fori_loop(0, 16, bs_idx, (jnp.int32(0), jnp.int32(_NP)))
    big = nvalid > _K
    elig_top = (key_i > vstar) | (emask & (idx2 >= tstar))
    elig = (big & elig_top) | (jnp.logical_not(big) & valid)
    eligf = elig.astype(jnp.float32)
    ef_ref[...] = eligf
    rk_ref[...] = _lane_prefix(eligf)

    zero48 = jnp.zeros((_DR, 128), jnp.float32)
    s_ref[...] = zero48
    x1_ref[...] = zero48
    y1_ref[...] = zero48
    x2_ref[...] = zero48
    y2_ref[...] = zero48
    idx_ref[...] = zero48
    fl_ref[...] = zero48

    def crow(r, w):
        rs = pl.ds(r, 1)
        lxr = lx_ref[rs, :]; lyr = ly_ref[rs, :]
        lwr = lw_ref[rs, :]; lhr = lh_ref[rs, :]
        pcxr = pcx_ref[rs, :]; pcyr = pcy_ref[rs, :]
        pwr = pw_ref[rs, :]; phr = ph_ref[rs, :]
        scr = sc_ref[rs, :]
        cxr = pcxr + (lxr * 0.1) * pwr
        cyr = pcyr + (lyr * 0.1) * phr
        wr_ = pwr * jnp.exp(lwr * 0.2)
        hr_ = phr * jnp.exp(lhr * 0.2)
        x1r = cxr - wr_ / 2.0
        y1r = cyr - hr_ / 2.0
        x2r = wr_ + x1r
        y2r = hr_ + y1r
        er = ef_ref[rs, :]
        rkr = rk_ref[rs, :]
        cnt = jnp.max(rkr)
        wf = w.astype(jnp.float32)
        lane = lax.broadcasted_iota(jnp.int32, (1, 128), 1).astype(jnp.float32)
        idxr = lane + lax.convert_element_type(r * 128, jnp.float32)
        g = jnp.where(er > 0.5, wf + (rkr - er), -1e9)
        gT = g.reshape(128, 1)
        q0 = w // 128
        base0 = lax.convert_element_type(q0 * 128, jnp.float32)
        P0 = (gT == lane + base0).astype(jnp.float32)
        P1 = (gT == lane + (base0 + 128.0)).astype(jnp.float32)
        D = jnp.concatenate(
            [scr, x1r, y1r, x2r, y2r, idxr, jnp.ones_like(scr),
             jnp.zeros_like(scr)], axis=0)
        B0 = lax.dot_general(D, P0, dimension_numbers=_DN,
                             precision=lax.Precision.HIGHEST)
        B1 = lax.dot_general(D, P1, dimension_numbers=_DN,
                             precision=lax.Precision.HIGHEST)
        q0s = pl.ds(q0, 1)
        q1s = pl.ds(q0 + 1, 1)
        s_ref[q0s, :] = s_ref[q0s, :] + B0[0:1]
        x1_ref[q0s, :] = x1_ref[q0s, :] + B0[1:2]
        y1_ref[q0s, :] = y1_ref[q0s, :] + B0[2:3]
        x2_ref[q0s, :] = x2_ref[q0s, :] + B0[3:4]
        y2_ref[q0s, :] = y2_ref[q0s, :] + B0[4:5]
        idx_ref[q0s, :] = idx_ref[q0s, :] + B0[5:6]
        fl_ref[q0s, :] = fl_ref[q0s, :] + B0[6:7]
        s_ref[q1s, :] = s_ref[q1s, :] + B1[0:1]
        x1_ref[q1s, :] = x1_ref[q1s, :] + B1[1:2]
        y1_ref[q1s, :] = y1_ref[q1s, :] + B1[2:3]
        x2_ref[q1s, :] = x2_ref[q1s, :] + B1[3:4]
        y2_ref[q1s, :] = y2_ref[q1s, :] + B1[4:5]
        idx_ref[q1s, :] = idx_ref[q1s, :] + B1[5:6]
        fl_ref[q1s, :] = fl_ref[q1s, :] + B1[6:7]
        return w + cnt.astype(jnp.int32)

    lax.fori_loop(0, _ROWS, crow, jnp.int32(0))

    real = fl_ref[...] > 0.5
    s0 = jnp.where(real, s_ref[...], _NEG)
    s_ref[...] = s0
    idx_ref[...] = jnp.where(real, idx_ref[...], -1.0)
    x1c = x1_ref[...]; y1c = y1_ref[...]
    x2c = x2_ref[...]; y2c = y2_ref[...]
    ar_ref[...] = (x2c - x1c) * (y2c - y1c)
    out_ref[...] = jnp.zeros((_TOPK, 1, 128), jnp.float32)

    def cond(carry):
        j, m = carry
        return (j < _TOPK) & (m > _NEG)

    def body(carry):
        j, m = carry
        s = s_ref[...]
        idxv = idx_ref[...]
        pick = jnp.max(jnp.where(s == m, idxv, -1.0))
        sel = idxv == pick
        bx1 = x1_ref[...]; by1 = y1_ref[...]
        bx2 = x2_ref[...]; by2 = y2_ref[...]
        ar = ar_ref[...]
        px1 = jnp.max(jnp.where(sel, bx1, _NEG))
        py1 = jnp.max(jnp.where(sel, by1, _NEG))
        px2 = jnp.max(jnp.where(sel, bx2, _NEG))
        py2 = jnp.max(jnp.where(sel, by2, _NEG))
        par = jnp.max(jnp.where(sel, ar, _NEG))
        xx1 = jnp.maximum(bx1, px1); yy1 = jnp.maximum(by1, py1)
        xx2 = jnp.minimum(bx2, px2); yy2 = jnp.minimum(by2, py2)
        ww = jnp.clip(xx2 - xx1, 0.0, None)
        hh = jnp.clip(yy2 - yy1, 0.0, None)
        inter = ww * hh
        union = ar - inter + par
        iou = inter / jnp.maximum(union, 1e-12)
        s_new = jnp.where((iou <= _THR) & (idxv != pick), s, _NEG)
        s_ref[...] = s_new
        c128 = lax.broadcasted_iota(jnp.int32, (1, 128), 1)
        row = jnp.where(c128 == 0, m, jnp.float32(0.0))
        row = jnp.where(c128 == 1, px1, row)
        row = jnp.where(c128 == 2, py1, row)
        row = jnp.where(c128 == 3, px2, row)
        row = jnp.where(c128 == 4, py2, row)
        out_ref[j] = row
        return (j + 1, jnp.max(s_new))

    m0 = jnp.max(s0)
    lax.while_loop(cond, body, (jnp.int32(0), m0))


_SCRATCHES = ([pltpu.VMEM((_ROWS, 128), jnp.float32)] * 2 +
              [pltpu.VMEM((_DR, 128), jnp.float32)] * 8)


def _run_nms(parts, interpret=False):
    return pl.pallas_call(
        _nms_body,
        out_shape=jax.ShapeDtypeStruct((_TOPK, 1, 128), jnp.float32),
        scratch_shapes=_SCRATCHES,
        interpret=interpret,
    )(*parts)


def _prep(loc_data, conf_data, prior_data):
    loc = jnp.pad(loc_data.reshape(_N, 4).T, ((0, 0), (0, _NP - _N)))
    pri = jnp.pad(prior_data.T, ((0, 0), (0, _NP - _N)))
    loc = loc.reshape(4, _ROWS, 128)
    pri = pri.reshape(4, _ROWS, 128)
    sc = jnp.pad(conf_data[:, 1], (0, _NP - _N)).reshape(_ROWS, 128)
    return [loc[0], loc[1], loc[2], loc[3], pri[0], pri[1], pri[2], pri[3], sc]


@jax.jit
def kernel(loc_data, conf_data, prior_data):
    out = _run_nms(_prep(loc_data, conf_data, prior_data))
    cls1 = out[:, 0, :5].reshape(1, 1, _TOPK, 5)
    zero = jnp.zeros((1, 1, _TOPK, 5), jnp.float32)
    return jnp.concatenate([zero, cls1], axis=1)
